# baseline (speedup 1.0000x reference)
import functools

import jax
import jax.numpy as jnp
from jax import lax
from jax.experimental import pallas as pl
from jax.experimental.pallas import tpu as pltpu
from jax.experimental.pallas import tpu_sc as plsc

_B = 128
NC, NS, LN = 2, 16, 16
NT = NC * NS
N = 50000
NPAD = 50176
SEG = NPAD // NS
WIN = NPAD // NT
E = 800000
EB = 128
BLOCKS = 196
EPT = BLOCKS * EB
EPAD = NT * EPT
NBLK = EPAD // EB
FC = 64
NB = 1024
GRID_N = NPAD // NB

_mesh = plsc.VectorSubcoreMesh(core_axis_name="c", subcore_axis_name="s",
                               num_cores=NC, num_subcores=NS)
_sc_params = pltpu.CompilerParams(use_tc_tiling_on_sc=False)
_sc_params_nl = pltpu.CompilerParams(use_tc_tiling_on_sc=False,
                                     needs_layout_passes=False)


def _deg_body(col_ref, valid_ref, out_ref, acc, cbuf, vbuf, zbuf, obuf, sem):
    c = lax.axis_index("c")
    s = lax.axis_index("s")
    for i in range(28):
        zbuf[pl.ds(i * 16, 16)] = jnp.zeros((16,), jnp.float32)
    base = s * SEG
    for i in range(7):
        pltpu.sync_copy(zbuf, acc.at[pl.ds(base + i * 448, 448)])
    plsc.subcore_barrier()
    tile_base = (c * NS + s) * EPT

    def blk(b, _):
        eb = tile_base + b * EB
        pltpu.sync_copy(col_ref.at[pl.ds(eb, EB)], cbuf)
        pltpu.sync_copy(valid_ref.at[pl.ds(eb, EB)], vbuf)
        pltpu.sync_copy(vbuf, acc.at[cbuf], add=True)
        return 0

    lax.fori_loop(0, BLOCKS, blk, 0)
    plsc.subcore_barrier()
    pltpu.sync_copy(acc.at[pl.ds(base, SEG)], obuf)
    pltpu.sync_copy(obuf, out_ref.at[pl.ds(c * NPAD + base, SEG)])


def _deg_kernel(col, valid):
    return pl.kernel(
        _deg_body,
        out_type=jax.ShapeDtypeStruct((NC * NPAD,), jnp.float32),
        mesh=_mesh,
        compiler_params=_sc_params,
        scratch_types=[
            pltpu.VMEM_SHARED((NPAD,), jnp.float32),
            pltpu.VMEM((EB,), jnp.int32),
            pltpu.VMEM((EB,), jnp.float32),
            pltpu.VMEM((448,), jnp.float32),
            pltpu.VMEM((SEG,), jnp.float32),
            pltpu.SemaphoreType.DMA,
        ],
    )(col, valid)


def _norm_body(perm_ref, row_ref, w_ref, cols_ref, dis_ref,
               rows_ref, norm_ref,
               pbuf, rbuf, cbuf, wbuf, dr, dc, nbuf, sem):
    c = lax.axis_index("c")
    s = lax.axis_index("s")
    tile_base = (c * NS + s) * EPT

    def blk(b, _):
        eb = tile_base + b * EB
        pltpu.sync_copy(perm_ref.at[pl.ds(eb, EB)], pbuf)
        pltpu.sync_copy(cols_ref.at[pl.ds(eb, EB)], cbuf)
        pltpu.make_async_copy(row_ref.at[pbuf], rbuf, sem).start()
        pltpu.make_async_copy(row_ref.at[pbuf], rbuf, sem).wait()
        pltpu.make_async_copy(w_ref.at[pbuf], wbuf, sem).start()
        pltpu.make_async_copy(w_ref.at[pbuf], wbuf, sem).wait()
        pltpu.make_async_copy(dis_ref.at[rbuf], dr, sem).start()
        pltpu.make_async_copy(dis_ref.at[rbuf], dr, sem).wait()
        pltpu.make_async_copy(dis_ref.at[cbuf], dc, sem).start()
        pltpu.make_async_copy(dis_ref.at[cbuf], dc, sem).wait()
        for g in range(8):
            sl = pl.ds(g * 16, 16)
            nbuf[sl] = dr[sl] * wbuf[sl] * dc[sl]
        pltpu.sync_copy(rbuf, rows_ref.at[pl.ds(eb, EB)])
        pltpu.sync_copy(nbuf, norm_ref.at[pl.ds(eb, EB)])
        return 0

    lax.fori_loop(0, BLOCKS, blk, 0)


def _norm_kernel(perm, row, w, cols, dis):
    return pl.kernel(
        _norm_body,
        out_type=[
            jax.ShapeDtypeStruct((EPAD,), jnp.int32),
            jax.ShapeDtypeStruct((EPAD,), jnp.float32),
        ],
        mesh=_mesh,
        compiler_params=_sc_params,
        scratch_types=[
            pltpu.VMEM((EB,), jnp.int32),
            pltpu.VMEM((EB,), jnp.int32),
            pltpu.VMEM((EB,), jnp.int32),
            pltpu.VMEM((EB,), jnp.float32),
            pltpu.VMEM((EB,), jnp.float32),
            pltpu.VMEM((EB,), jnp.float32),
            pltpu.VMEM((EB,), jnp.float32),
            pltpu.SemaphoreType.DMA,
        ],
    )(perm, row, w, cols, dis)


def _agg_body(nchunks, xc_ref, rows_ref, cols_ref, norm_ref, meta_ref,
              out_ref, lbuf, rbuf, rbuf2, cbuf, nbuf, gbuf, mbuf, sem):
    c = lax.axis_index("c")
    s = lax.axis_index("s")
    t = c * NS + s
    pltpu.sync_copy(meta_ref.at[0, t], mbuf)
    start_blk = jnp.max(mbuf[...])
    pltpu.sync_copy(meta_ref.at[1, t], mbuf)
    nblk = jnp.max(mbuf[...])
    w0 = t * WIN

    for k in range(nchunks):
        def zero(i, _):
            for q in range(FC // 16):
                lbuf[i, pl.ds(q * 16, 16)] = jnp.zeros((16,), jnp.float32)
            return 0

        lax.fori_loop(0, WIN, zero, 0)

        def blk(b, _):
            eb = (start_blk + b) * EB
            pltpu.sync_copy(rows_ref.at[pl.ds(eb, EB)], rbuf)
            pltpu.sync_copy(cols_ref.at[pl.ds(eb, EB)], cbuf)
            pltpu.sync_copy(norm_ref.at[pl.ds(eb, EB)], nbuf)
            for g in range(8):
                sl = pl.ds(g * 16, 16)
                rbuf2[sl] = rbuf[sl] + (k * NPAD)
            pltpu.make_async_copy(xc_ref.at[rbuf2], gbuf, sem).start()
            pltpu.make_async_copy(xc_ref.at[rbuf2], gbuf, sem).wait()

            def egrp(g, _):
                cv = cbuf[pl.ds(g * 16, 16)]
                nv = nbuf[pl.ds(g * 16, 16)]
                inw = (cv >= w0) & (cv < w0 + WIN)
                nv = jnp.where(inw, nv, 0.0)
                lidx = jnp.clip(cv - w0, 0, WIN - 1)
                for l in range(16):
                    li = lidx[l]
                    nval = nv[l]
                    e = g * 16 + l
                    for q in range(FC // 16):
                        sl = pl.ds(q * 16, 16)
                        lbuf[li, sl] = lbuf[li, sl] + gbuf[e, sl] * nval
                return 0

            lax.fori_loop(0, EB // 16, egrp, 0)
            return 0

        lax.fori_loop(0, nblk, blk, 0)
        pltpu.sync_copy(lbuf, out_ref.at[pl.ds(k * NPAD + w0, WIN)])


def _agg_kernel(xc, rows, cols, norm, meta, nchunks):
    return pl.kernel(
        functools.partial(_agg_body, nchunks),
        out_type=jax.ShapeDtypeStruct((nchunks * NPAD, FC), jnp.float32),
        mesh=_mesh,
        compiler_params=_sc_params_nl,
        scratch_types=[
            pltpu.VMEM((WIN, FC), jnp.float32),
            pltpu.VMEM((EB,), jnp.int32),
            pltpu.VMEM((EB,), jnp.int32),
            pltpu.VMEM((EB,), jnp.int32),
            pltpu.VMEM((EB,), jnp.float32),
            pltpu.VMEM((EB, FC), jnp.float32),
            pltpu.VMEM((16,), jnp.int32),
            pltpu.SemaphoreType.DMA,
        ],
    )(xc, rows, cols, norm, meta)


def _mat_body(c1, c2, x_ref, w_ref, out_ref):
    acc = None
    for k in range(c1):
        t = jnp.dot(x_ref[k], w_ref[k], preferred_element_type=jnp.float32)
        acc = t if acc is None else acc + t
    for k2 in range(c2):
        out_ref[k2] = acc[:, k2 * FC:(k2 + 1) * FC]


def _mat_kernel(xc, wc):
    c1 = xc.shape[0]
    c2 = wc.shape[2] // FC
    return pl.pallas_call(
        functools.partial(_mat_body, c1, c2),
        grid=(GRID_N,),
        in_specs=[
            pl.BlockSpec((c1, NB, FC), lambda i: (0, i, 0)),
            pl.BlockSpec((c1, FC, c2 * FC), lambda i: (0, 0, 0)),
        ],
        out_specs=pl.BlockSpec((c2, NB, FC), lambda i: (0, i, 0)),
        out_shape=jax.ShapeDtypeStruct((c2, NPAD, FC), jnp.float32),
    )(xc, wc)


def _fuse_mat_body(c1, c2, s_ref, h_ref, dinv_ref, w_ref, out_ref):
    dinv = dinv_ref[...]
    acc = None
    for k in range(c1):
        xk = jnp.maximum(s_ref[k] + h_ref[k] * dinv, 0.0)
        t = jnp.dot(xk, w_ref[k], preferred_element_type=jnp.float32)
        acc = t if acc is None else acc + t
    for k2 in range(c2):
        out_ref[k2] = acc[:, k2 * FC:(k2 + 1) * FC]


def _fuse_mat_kernel(s3, h, dinv, wc):
    c1 = h.shape[0]
    c2 = wc.shape[2] // FC
    return pl.pallas_call(
        functools.partial(_fuse_mat_body, c1, c2),
        grid=(GRID_N,),
        in_specs=[
            pl.BlockSpec((c1, NB, FC), lambda i: (0, i, 0)),
            pl.BlockSpec((c1, NB, FC), lambda i: (0, i, 0)),
            pl.BlockSpec((NB, 1), lambda i: (i, 0)),
            pl.BlockSpec((c1, FC, c2 * FC), lambda i: (0, 0, 0)),
        ],
        out_specs=pl.BlockSpec((c2, NB, FC), lambda i: (0, i, 0)),
        out_shape=jax.ShapeDtypeStruct((c2, NPAD, FC), jnp.float32),
    )(s3, h, dinv, wc)


def _finish_body(c1, s_ref, h_ref, dinv_ref, out_ref):
    dinv = dinv_ref[...]
    for k in range(c1):
        out_ref[k] = jnp.maximum(s_ref[k] + h_ref[k] * dinv, 0.0)


def _finish_kernel(s3, h, dinv):
    c1 = h.shape[0]
    return pl.pallas_call(
        functools.partial(_finish_body, c1),
        grid=(GRID_N,),
        in_specs=[
            pl.BlockSpec((c1, NB, FC), lambda i: (0, i, 0)),
            pl.BlockSpec((c1, NB, FC), lambda i: (0, i, 0)),
            pl.BlockSpec((NB, 1), lambda i: (i, 0)),
        ],
        out_specs=pl.BlockSpec((c1, NB, FC), lambda i: (0, i, 0)),
        out_shape=jax.ShapeDtypeStruct((c1, NPAD, FC), jnp.float32),
    )(s3, h, dinv)


_PB = 224


def _pool_body(c3, x_ref, batch_ref, valid_ref, segp_ref, cntp_ref,
               abuf, cbuf2, xbuf, bbuf, vbuf, sem):
    c = lax.axis_index("c")
    s = lax.axis_index("s")
    t = c * NS + s
    r0 = t * WIN

    def zcnt(i, _):
        cbuf2[i, pl.ds(0, 16)] = jnp.zeros((16,), jnp.float32)
        return 0

    lax.fori_loop(0, _B, zcnt, 0)

    for k in range(c3):
        def zacc(i, _):
            for q in range(FC // 16):
                abuf[i, pl.ds(q * 16, 16)] = jnp.zeros((16,), jnp.float32)
            return 0

        lax.fori_loop(0, _B, zacc, 0)

        def blk(j, _):
            rbase = k * NPAD + r0 + j * _PB
            pltpu.sync_copy(x_ref.at[pl.ds(rbase, _PB)], xbuf)
            pltpu.sync_copy(batch_ref.at[pl.ds(r0 + j * _PB, _PB)], bbuf)
            if k == 0:
                pltpu.sync_copy(valid_ref.at[pl.ds(r0 + j * _PB, _PB)], vbuf)

            def grp(g, _):
                bv = bbuf[pl.ds(g * 16, 16)]
                vv = vbuf[pl.ds(g * 16, 16)]
                for l in range(16):
                    bid = bv[l]
                    r = g * 16 + l
                    for q in range(FC // 16):
                        sl = pl.ds(q * 16, 16)
                        abuf[bid, sl] = abuf[bid, sl] + xbuf[r, sl]
                    if k == 0:
                        cbuf2[bid, pl.ds(0, 16)] = (
                            cbuf2[bid, pl.ds(0, 16)] + vv[l])
                return 0

            lax.fori_loop(0, _PB // 16, grp, 0)
            return 0

        lax.fori_loop(0, WIN // _PB, blk, 0)
        pltpu.sync_copy(abuf, segp_ref.at[pl.ds((t * c3 + k) * _B, _B)])

    pltpu.sync_copy(cbuf2, cntp_ref.at[pl.ds(t * _B, _B)])


def _pool_sc_kernel(x3, batch, valid):
    c3 = x3.shape[0]
    return pl.kernel(
        functools.partial(_pool_body, c3),
        out_type=[
            jax.ShapeDtypeStruct((NT * c3 * _B, FC), jnp.float32),
            jax.ShapeDtypeStruct((NT * _B, 16), jnp.float32),
        ],
        mesh=_mesh,
        compiler_params=_sc_params,
        scratch_types=[
            pltpu.VMEM((_B, FC), jnp.float32),
            pltpu.VMEM((_B, 16), jnp.float32),
            pltpu.VMEM((_PB, FC), jnp.float32),
            pltpu.VMEM((_PB,), jnp.int32),
            pltpu.VMEM((_PB,), jnp.float32),
            pltpu.SemaphoreType.DMA,
        ],
    )(x3.reshape(c3 * NPAD, FC), batch, valid)


def _head_body(cd, cp, dsegp, dcntp, psegp, pcntp,
               dL1w, dL1b, dL2w, dL2b, pL1w, pL1b, pL2w, pL2b,
               fW1, fb1, fW2, fb2, fW3, fb3, out_ref):
    relu = lambda v: jnp.maximum(v, 0.0)

    def branch(segp, cntp, w1, b1, w2, b2, chunks):
        seg = jnp.sum(segp[...].reshape(NT, chunks, _B, FC), axis=0)
        cnt = jnp.sum(cntp[...].reshape(NT, _B, 16), axis=0)[:, 0:1]
        inv = 1.0 / jnp.maximum(cnt, 1.0)
        acc = None
        for k in range(chunks):
            t = jnp.dot(seg[k] * inv, w1[k],
                        preferred_element_type=jnp.float32)
            acc = t if acc is None else acc + t
        h = relu(acc + b1[...])
        return relu(jnp.dot(h, w2[...], preferred_element_type=jnp.float32)
                    + b2[...])

    x = branch(dsegp, dcntp, dL1w, dL1b, dL2w, dL2b, cd)
    p = branch(psegp, pcntp, pL1w, pL1b, pL2w, pL2b, cp)
    cvec = jnp.concatenate([x, p], axis=1)
    h = relu(jnp.dot(cvec, fW1[...], preferred_element_type=jnp.float32)
             + fb1[...])
    h = relu(jnp.dot(h, fW2[...], preferred_element_type=jnp.float32)
             + fb2[...])
    out_ref[...] = (jnp.dot(h, fW3[...], preferred_element_type=jnp.float32)
                    + fb3[...])


def _head_kernel(dseg, dcnt, pseg, pcnt, params):
    cd = dseg.shape[0] // (NT * _B)
    cp = pseg.shape[0] // (NT * _B)
    dL1w = _chunk_w(_pad_rows(params['dL1_w'], cd * FC))
    pL1w = _chunk_w(_pad_rows(params['pL1_w'], cp * FC))
    args = (dseg, dcnt, pseg, pcnt,
            dL1w, params['dL1_b'].reshape(1, -1),
            params['dL2_w'], params['dL2_b'].reshape(1, -1),
            pL1w, params['pL1_b'].reshape(1, -1),
            params['pL2_w'], params['pL2_b'].reshape(1, -1),
            params['fW1'], params['fb1'].reshape(1, -1),
            params['fW2'], params['fb2'].reshape(1, -1),
            params['fW3'], params['fb3'].reshape(1, -1))
    out = pl.pallas_call(
        functools.partial(_head_body, cd, cp),
        out_shape=jax.ShapeDtypeStruct((_B, 1), jnp.float32),
    )(*args)
    return out[:, 0]


def _cdiv(a, b):
    return (a + b - 1) // b


def _pad_rows(w, rows):
    return jnp.pad(w, ((0, rows - w.shape[0]), (0, 0)))


def _chunk_w(w):
    rows, cols = w.shape
    return w.reshape(rows // FC, FC, cols)


def _pack_x(x):
    n, f = x.shape
    c = _cdiv(f, FC)
    xp = jnp.pad(x, ((0, NPAD - n), (0, c * FC - f)))
    return xp.reshape(NPAD, c, FC).transpose(1, 0, 2)


def _gcn_stack(x, edge_index, edge_attr, batch, w1, w2, w3):
    row = jnp.pad(edge_index[0], (0, EPAD - E))
    col = jnp.pad(edge_index[1], (0, EPAD - E), constant_values=NPAD - 1)
    wv = jnp.pad(edge_attr.reshape(-1), (0, EPAD - E))
    validv = jnp.pad(jnp.ones((E,), jnp.float32), (0, EPAD - E))

    partial_deg = _deg_kernel(col, validv).reshape(NC, NPAD)
    deg = 1.0 + partial_deg[0] + partial_deg[1]
    dis = deg ** -0.5
    dinv = (dis * dis)[:, None]

    perm = jnp.argsort(col).astype(jnp.int32)
    col_s = col[perm]
    bnds = jnp.searchsorted(col_s, jnp.arange(NT + 1, dtype=jnp.int32) * WIN
                            ).astype(jnp.int32)
    start_blk = bnds[:-1] // EB
    end_blk = -((-bnds[1:]) // EB)
    meta = jnp.stack([start_blk, end_blk - start_blk])
    meta = jnp.broadcast_to(meta[:, :, None], (2, NT, 16)).astype(jnp.int32)

    row_s, norm_s = _norm_kernel(perm, row, wv, col_s, dis)

    xc = _pack_x(x)
    ws = []
    fin = x.shape[1]
    for w in (w1, w2, w3):
        c1 = _cdiv(fin, FC)
        c2 = _cdiv(w.shape[1], FC)
        wp = jnp.pad(w, ((0, c1 * FC - w.shape[0]), (0, c2 * FC - w.shape[1])))
        ws.append(wp.reshape(c1, FC, c2 * FC))
        fin = w.shape[1]

    h = _mat_kernel(xc, ws[0])
    for wc in (ws[1], ws[2]):
        ch = h.shape[0]
        s_flat = _agg_kernel(h.reshape(ch * NPAD, FC), row_s, col_s, norm_s,
                             meta, ch)
        s3 = s_flat.reshape(ch, NPAD, FC)
        h = _fuse_mat_kernel(s3, h, dinv, wc)
    ch = h.shape[0]
    s_flat = _agg_kernel(h.reshape(ch * NPAD, FC), row_s, col_s, norm_s,
                         meta, ch)
    x3 = _finish_kernel(s_flat.reshape(ch, NPAD, FC), h, dinv)

    batch_p = jnp.pad(batch, (0, NPAD - N)).astype(jnp.int32)
    valid_n = jnp.pad(jnp.ones((N,), jnp.float32), (0, NPAD - N))
    return _pool_sc_kernel(x3, batch_p, valid_n)


def kernel(drug_x, drug_edge_index, drug_edge_attr, drug_batch,
           protein_x, protein_edge_index, protein_edge_attr, protein_batch,
           params):
    dseg, dcnt = _gcn_stack(drug_x, drug_edge_index, drug_edge_attr,
                            drug_batch, params['dW1'], params['dW2'],
                            params['dW3'])
    pseg, pcnt = _gcn_stack(protein_x, protein_edge_index, protein_edge_attr,
                            protein_batch, params['pW1'], params['pW2'],
                            params['pW3'])
    return _head_kernel(dseg, dcnt, pseg, pcnt, params)
